# Initial kernel scaffold; baseline (speedup 1.0000x reference)
import jax
import jax.numpy as jnp
from jax.experimental import pallas as pl


def kernel(x, edge_index, edge_attr, Wm1, bm1, We1, be1, Ws1, bs1, Wm2, bm2, We2, be2, Ws2, bs2):
    raise NotImplementedError("write your pallas kernel here")


# speedup vs baseline: 4.5106x; 4.5106x over previous
import functools

import jax
import jax.numpy as jnp
from jax import lax
from jax.experimental import pallas as pl
from jax.experimental.pallas import tpu as pltpu
from jax.experimental.pallas import tpu_sc as plsc

_CHUNK = 128
_BN = 1024
_NP = 10240


def _sc_segsum(n_pad, width, n_edges, gather):
  info = plsc.get_sparse_core_info()
  nc, ns = info.num_cores, info.num_subcores
  nw = nc * ns
  n_chunks = n_edges // _CHUNK
  assert n_edges % _CHUNK == 0
  rows_sub = n_pad // ns
  assert n_pad % (ns * 8) == 0
  zrows = 32
  assert rows_sub % zrows == 0

  mesh = plsc.VectorSubcoreMesh(core_axis_name="c", subcore_axis_name="s")
  out_type = [jax.ShapeDtypeStruct((nc, n_pad, width), jnp.float32)]
  scratch = [
      pltpu.VMEM((_CHUNK,), jnp.int32),
      pltpu.VMEM((_CHUNK, width), jnp.float32),
      pltpu.VMEM((zrows, width), jnp.float32),
      pltpu.VMEM_SHARED((n_pad, width), jnp.float32),
      pltpu.SemaphoreType.DMA,
  ]
  if gather:
    scratch = [pltpu.VMEM((_CHUNK,), jnp.int32)] + scratch
  else:
    out_type.append(jax.ShapeDtypeStruct((nw * n_pad,), jnp.float32))
    scratch.append(pltpu.VMEM((n_pad,), jnp.float32))

  @functools.partial(
      pl.kernel,
      out_type=out_type[0] if gather else tuple(out_type),
      mesh=mesh,
      scratch_types=tuple(scratch),
      compiler_params=pltpu.CompilerParams(needs_layout_passes=False),
  )
  def body(*refs):
    if gather:
      (table, src, dst, out, idx_s, idx_d, buf, zbuf, accum, sem) = refs
    else:
      (rows, dst, out, deg_out, idx_d, buf, zbuf, accum, sem, deg) = refs
    c = lax.axis_index("c")
    s = lax.axis_index("s")
    w = s * nc + c
    row0 = s * rows_sub

    for i in range(zrows):
      for j in range(width // 16):
        zbuf[i, pl.ds(j * 16, 16)] = jnp.zeros((16,), jnp.float32)
    for t in range(rows_sub // zrows):
      pltpu.sync_copy(zbuf, accum.at[pl.ds(row0 + t * zrows, zrows), :])
    if not gather:
      def zdeg(t, _):
        deg[pl.ds(t * 16, 16)] = jnp.zeros((16,), jnp.float32)
        return 0
      lax.fori_loop(0, n_pad // 16, zdeg, 0)
    plsc.subcore_barrier()

    extra = n_chunks % nw
    n_i = (n_chunks // nw) + jnp.where(w < extra, 1, 0)
    ones16 = jnp.ones((16,), jnp.float32)

    def step(i, _):
      base = (w + i * nw) * _CHUNK
      pltpu.sync_copy(dst.at[pl.ds(base, _CHUNK)], idx_d)
      if gather:
        pltpu.sync_copy(src.at[pl.ds(base, _CHUNK)], idx_s)
        pltpu.async_copy(table.at[idx_s], buf, sem).wait()
      else:
        pltpu.sync_copy(rows.at[pl.ds(base, _CHUNK), :], buf)
        for j in range(_CHUNK // 16):
          plsc.addupdate_scatter(deg, [idx_d[pl.ds(j * 16, 16)]], ones16)
      pltpu.sync_copy(buf, accum.at[idx_d], add=True)
      return 0

    lax.fori_loop(0, n_i, step, 0)
    plsc.subcore_barrier()
    pltpu.sync_copy(accum.at[pl.ds(row0, rows_sub), :],
                    out.at[c, pl.ds(row0, rows_sub), :])
    if not gather:
      pltpu.sync_copy(deg, deg_out.at[pl.ds(w * n_pad, n_pad)])

  return body


def _tc_combine(n_pad, relu):
  grid = n_pad // _BN

  def body(g_ref, a_ref, x_ref, dp_ref, wm_ref, we_ref, ws_ref, dm_ref,
           b_ref, o_ref):
    g = g_ref[0] + g_ref[1]
    a = a_ref[0] + a_ref[1]
    t = (jnp.dot(g, wm_ref[...], preferred_element_type=jnp.float32)
         + jnp.dot(a, we_ref[...], preferred_element_type=jnp.float32)
         + jnp.dot(x_ref[...], ws_ref[...], preferred_element_type=jnp.float32)
         + lax.dot_general(dp_ref[...], dm_ref[...],
                           (((0,), (0,)), ((), ())),
                           preferred_element_type=jnp.float32)
         + b_ref[...])
    if relu:
      t = jnp.maximum(t, 0.0)
    o_ref[...] = t

  nw = 32
  return pl.pallas_call(
      body,
      grid=(grid,),
      in_specs=[
          pl.BlockSpec((2, _BN, 128), lambda i: (0, i, 0)),
          pl.BlockSpec((2, _BN, 128), lambda i: (0, i, 0)),
          pl.BlockSpec((_BN, 128), lambda i: (i, 0)),
          pl.BlockSpec((nw, _BN), lambda i: (0, i)),
          pl.BlockSpec((128, 128), lambda i: (0, 0)),
          pl.BlockSpec((128, 128), lambda i: (0, 0)),
          pl.BlockSpec((128, 128), lambda i: (0, 0)),
          pl.BlockSpec((nw, 128), lambda i: (0, 0)),
          pl.BlockSpec((1, 128), lambda i: (0, 0)),
      ],
      out_specs=pl.BlockSpec((_BN, 128), lambda i: (i, 0)),
      out_shape=jax.ShapeDtypeStruct((n_pad, 128), jnp.float32),
  )


def kernel(x, edge_index, edge_attr, Wm1, bm1, We1, be1, Ws1, bs1,
           Wm2, bm2, We2, be2, Ws2, bs2):
  n, d = x.shape
  e = edge_index.shape[1]
  nw = 32
  src = edge_index[0]
  dst = edge_index[1]

  xp = jnp.concatenate([x, jnp.zeros((_NP - n, d), jnp.float32)], axis=0)
  dm1 = jnp.broadcast_to(bm1 + be1, (nw, d))
  dm2 = jnp.broadcast_to(bm2 + be2, (nw, d))

  seg_lin = _sc_segsum(_NP, d, e, gather=False)
  seg_gat = _sc_segsum(_NP, d, e, gather=True)
  layer1 = _tc_combine(_NP, relu=True)
  layer2 = _tc_combine(_NP, relu=False)

  a_parts, deg1d = seg_lin(edge_attr, dst)
  degp = deg1d.reshape(nw, _NP)
  g1_parts = seg_gat(xp, src, dst)
  h = layer1(g1_parts, a_parts, xp, degp, Wm1, We1, Ws1, dm1, bs1[None, :])
  g2_parts = seg_gat(h, src, dst)
  out = layer2(g2_parts, a_parts, h, degp, Wm2, We2, Ws2, dm2, bs2[None, :])
  return (out[:n], edge_index, edge_attr)
